# baseline (speedup 1.0000x reference)
import functools

import jax
import jax.numpy as jnp
from jax import lax
from jax.experimental import pallas as pl
from jax.experimental.pallas import tpu as pltpu
from jax.experimental.pallas import tpu_sc as plsc

_PERIOD = 4
_START = 1
_NC = 2
_NS = 16
_NW = _NC * _NS

_CHUNK = 8192


def _decimate_body(x_hbm, o_hbm, idx0, idx1, d0, d1, sem_g0, sem_g1,
                   sem_out0, sem_out1, *, n_out):
    wid = lax.axis_index("s") * _NC + lax.axis_index("c")
    per_w = n_out // _NW
    base_out = wid * per_w
    n_chunks = per_w // _CHUNK
    iota16 = lax.iota(jnp.int32, 16)

    idx = (idx0, idx1)
    data = (d0, d1)
    sem_gs = (sem_g0, sem_g1)
    sem_outs = (sem_out0, sem_out1)

    base4 = base_out * _PERIOD + _START

    def fill_body(i, _):
        v = iota16 * _PERIOD + (base4 + i * 16 * _PERIOD)
        idx0[pl.ds(i * 16, 16)] = v
        idx1[pl.ds(i * 16, 16)] = v + _CHUNK * _PERIOD
        return 0

    lax.fori_loop(0, _CHUNK // 16, fill_body, 0, unroll=8)

    def advance(b):
        def body(i, _):
            idx[b][pl.ds(i * 16, 16)] = (
                idx[b][pl.ds(i * 16, 16)] + 2 * _CHUNK * _PERIOD)
            return 0

        lax.fori_loop(0, _CHUNK // 16, body, 0, unroll=8)

    def gather(c):
        return pltpu.async_copy(x_hbm.at[idx[c % 2]], data[c % 2],
                                sem_gs[c % 2])

    def wait_gather(c):
        pltpu.make_async_copy(x_hbm.at[idx[c % 2]], data[c % 2],
                              sem_gs[c % 2]).wait()

    def out_copy(c):
        off = base_out + c * _CHUNK
        return pltpu.async_copy(
            data[c % 2], o_hbm.at[pl.ds(off, _CHUNK)], sem_outs[c % 2])

    gather(0)
    gather(1)
    for c in range(n_chunks):
        wait_gather(c)
        if c + 2 < n_chunks:
            advance(c % 2)
        if 1 <= c and c + 1 < n_chunks:
            pltpu.make_async_copy(
                data[(c + 1) % 2], o_hbm.at[pl.ds(0, _CHUNK)],
                sem_outs[(c + 1) % 2]).wait()
            gather(c + 1)
        out_copy(c)
    for c in (n_chunks - 2, n_chunks - 1):
        pltpu.make_async_copy(
            data[c % 2], o_hbm.at[pl.ds(0, _CHUNK)], sem_outs[c % 2]).wait()


@functools.partial(jax.jit, static_argnums=(1,))
def _decimate_flat(x_flat, n_out):
    body = functools.partial(_decimate_body, n_out=n_out)
    return pl.kernel(
        body,
        out_type=jax.ShapeDtypeStruct((n_out,), jnp.float32),
        mesh=plsc.VectorSubcoreMesh(core_axis_name="c", subcore_axis_name="s"),
        scratch_types=[
            pltpu.VMEM((_CHUNK,), jnp.int32),
            pltpu.VMEM((_CHUNK,), jnp.int32),
            pltpu.VMEM((_CHUNK,), jnp.float32),
            pltpu.VMEM((_CHUNK,), jnp.float32),
            pltpu.SemaphoreType.DMA,
            pltpu.SemaphoreType.DMA,
            pltpu.SemaphoreType.DMA,
            pltpu.SemaphoreType.DMA,
        ],
        compiler_params=pltpu.CompilerParams(needs_layout_passes=False),
    )(x_flat)


def kernel(x):
    shape = x.shape
    t = shape[-1]
    assert t % _PERIOD == 0
    n_out_t = t // _PERIOD
    n_out = x.size // _PERIOD
    y = _decimate_flat(x.reshape(-1), n_out)
    return y.reshape(*shape[:-1], n_out_t)


# speedup vs baseline: 1.3303x; 1.3303x over previous
import functools

import jax
import jax.numpy as jnp
from jax import lax
from jax.experimental import pallas as pl
from jax.experimental.pallas import tpu as pltpu
from jax.experimental.pallas import tpu_sc as plsc

_PERIOD = 4
_START = 1
_NC = 2
_NS = 16
_NW = _NC * _NS

_CHUNK = 8192
_K_SC = 1024
_TC_R = 16


def _sc_body(x_hbm, o_hbm, in_v0, in_v1, out_v0, out_v1,
             sem_in0, sem_in1, sem_out0, sem_out1, *, n_out_sc):
    wid = lax.axis_index("s") * _NC + lax.axis_index("c")
    per_w = n_out_sc // _NW
    base_out = wid * per_w
    n_chunks = per_w // _CHUNK
    idx0 = lax.iota(jnp.int32, 16) * _PERIOD + _START

    in_bufs = (in_v0, in_v1)
    out_bufs = (out_v0, out_v1)
    sem_ins = (sem_in0, sem_in1)
    sem_outs = (sem_out0, sem_out1)

    def in_copy(c):
        off = (base_out + c * _CHUNK) * _PERIOD
        return pltpu.async_copy(
            x_hbm.at[pl.ds(off, _CHUNK * _PERIOD)], in_bufs[c % 2],
            sem_ins[c % 2])

    def out_copy(c):
        off = base_out + c * _CHUNK
        return pltpu.async_copy(
            out_bufs[c % 2], o_hbm.at[pl.ds(off, _CHUNK)], sem_outs[c % 2])

    in_copy(0)
    for c in range(n_chunks):
        b = c % 2
        pltpu.make_async_copy(
            x_hbm.at[pl.ds(0, _CHUNK * _PERIOD)], in_bufs[b],
            sem_ins[b]).wait()
        if c + 1 < n_chunks:
            in_copy(c + 1)
        if c >= 2:
            pltpu.make_async_copy(
                out_bufs[b], o_hbm.at[pl.ds(0, _CHUNK)], sem_outs[b]).wait()

        def vec_body(i, _, b=b):
            out_bufs[b][pl.ds(i * 16, 16)] = plsc.load_gather(
                in_bufs[b], [idx0 + i * (16 * _PERIOD)])
            return 0

        lax.fori_loop(0, _CHUNK // 16, vec_body, 0, unroll=8)
        out_copy(c)
    for c in (n_chunks - 2, n_chunks - 1):
        pltpu.make_async_copy(
            out_bufs[c % 2], o_hbm.at[pl.ds(0, _CHUNK)],
            sem_outs[c % 2]).wait()


def _tc_body(x_ref, sel_ref, o_ref):
    r, p, l = x_ref.shape
    x2 = x_ref[...].reshape(r * p, l)
    y = jnp.dot(x2, sel_ref[...], preferred_element_type=jnp.float32,
                precision=lax.Precision.HIGHEST)
    o_ref[...] = y.reshape(r, p, l // _PERIOD)


@functools.partial(jax.jit, static_argnums=(1, 2))
def _decimate(x2d, n_rows, t):
    n_out_t = t // _PERIOD
    n_out_sc = _K_SC * n_out_t
    sel = (lax.broadcasted_iota(jnp.int32, (128, 128 // _PERIOD), 0) ==
           lax.broadcasted_iota(jnp.int32, (128, 128 // _PERIOD), 1) *
           _PERIOD + _START).astype(jnp.float32)

    sc_body = functools.partial(_sc_body, n_out_sc=n_out_sc)
    y_sc = pl.kernel(
        sc_body,
        out_type=jax.ShapeDtypeStruct((n_rows * n_out_t,), jnp.float32),
        mesh=plsc.VectorSubcoreMesh(core_axis_name="c", subcore_axis_name="s"),
        scratch_types=[
            pltpu.VMEM((_CHUNK * _PERIOD,), jnp.float32),
            pltpu.VMEM((_CHUNK * _PERIOD,), jnp.float32),
            pltpu.VMEM((_CHUNK,), jnp.float32),
            pltpu.VMEM((_CHUNK,), jnp.float32),
            pltpu.SemaphoreType.DMA,
            pltpu.SemaphoreType.DMA,
            pltpu.SemaphoreType.DMA,
            pltpu.SemaphoreType.DMA,
        ],
        compiler_params=pltpu.CompilerParams(needs_layout_passes=False),
    )(x2d.reshape(-1))

    n_tc_rows = n_rows - _K_SC
    n_p = t // 128
    y_tc = pl.pallas_call(
        _tc_body,
        grid=(n_tc_rows // _TC_R,),
        in_specs=[
            pl.BlockSpec((_TC_R, n_p, 128), lambda i: (i + _K_SC // _TC_R, 0, 0)),
            pl.BlockSpec((128, 128 // _PERIOD), lambda i: (0, 0)),
        ],
        out_specs=pl.BlockSpec((_TC_R, n_p, 128 // _PERIOD),
                               lambda i: (i, 0, 0)),
        out_shape=jax.ShapeDtypeStruct((n_tc_rows, n_p, 128 // _PERIOD),
                                       jnp.float32),
    )(x2d.reshape(n_rows, n_p, 128), sel)

    y = lax.dynamic_update_slice(
        y_sc.reshape(n_rows, n_out_t), y_tc.reshape(n_tc_rows, n_out_t),
        (_K_SC, 0))
    return y


def kernel(x):
    shape = x.shape
    t = shape[-1]
    assert t % (_PERIOD * 128) == 0
    n_rows = x.size // t
    y = _decimate(x.reshape(n_rows, t), n_rows, t)
    return y.reshape(*shape[:-1], t // _PERIOD)


# speedup vs baseline: 2.3000x; 1.7290x over previous
import functools

import jax
import jax.numpy as jnp
from jax import lax
from jax.experimental import pallas as pl
from jax.experimental.pallas import tpu as pltpu
from jax.experimental.pallas import tpu_sc as plsc

_PERIOD = 4
_START = 1
_NC = 2
_NS = 16
_NW = _NC * _NS

_CHUNK = 8192


def _decimate_body(x_hbm, o_hbm, in_v0, in_v1, out_v0, out_v1,
                   sem_in0, sem_in1, sem_out0, sem_out1, *, n_out):
    wid = lax.axis_index("s") * _NC + lax.axis_index("c")
    per_w = n_out // _NW
    base_out = wid * per_w
    n_chunks = per_w // _CHUNK
    idx0 = lax.iota(jnp.int32, 16) * _PERIOD + _START

    in_bufs = (in_v0, in_v1)
    out_bufs = (out_v0, out_v1)
    sem_ins = (sem_in0, sem_in1)
    sem_outs = (sem_out0, sem_out1)

    def in_copy(c):
        off = (base_out + c * _CHUNK) * _PERIOD
        return pltpu.async_copy(
            x_hbm.at[pl.ds(off, _CHUNK * _PERIOD)], in_bufs[c % 2],
            sem_ins[c % 2])

    def out_copy(c):
        off = base_out + c * _CHUNK
        return pltpu.async_copy(
            out_bufs[c % 2], o_hbm.at[pl.ds(off, _CHUNK)], sem_outs[c % 2])

    in_copy(0)
    for c in range(n_chunks):
        b = c % 2
        pltpu.make_async_copy(
            x_hbm.at[pl.ds(0, _CHUNK * _PERIOD)], in_bufs[b],
            sem_ins[b]).wait()
        if c + 1 < n_chunks:
            in_copy(c + 1)
        if c >= 2:
            pltpu.make_async_copy(
                out_bufs[b], o_hbm.at[pl.ds(0, _CHUNK)], sem_outs[b]).wait()

        def vec_body(i, _, b=b):
            out_bufs[b][pl.ds(i * 16, 16)] = plsc.load_gather(
                in_bufs[b], [idx0 + i * (16 * _PERIOD)])
            return 0

        lax.fori_loop(0, _CHUNK // 16, vec_body, 0, unroll=8)
        out_copy(c)
    for c in (n_chunks - 2, n_chunks - 1):
        pltpu.make_async_copy(
            out_bufs[c % 2], o_hbm.at[pl.ds(0, _CHUNK)],
            sem_outs[c % 2]).wait()


@functools.partial(jax.jit, static_argnums=(1,))
def _decimate_flat(x_flat, n_out):
    body = functools.partial(_decimate_body, n_out=n_out)
    return pl.kernel(
        body,
        out_type=jax.ShapeDtypeStruct((n_out,), jnp.float32),
        mesh=plsc.VectorSubcoreMesh(core_axis_name="c", subcore_axis_name="s"),
        scratch_types=[
            pltpu.VMEM((_CHUNK * _PERIOD,), jnp.float32),
            pltpu.VMEM((_CHUNK * _PERIOD,), jnp.float32),
            pltpu.VMEM((_CHUNK,), jnp.float32),
            pltpu.VMEM((_CHUNK,), jnp.float32),
            pltpu.SemaphoreType.DMA,
            pltpu.SemaphoreType.DMA,
            pltpu.SemaphoreType.DMA,
            pltpu.SemaphoreType.DMA,
        ],
        compiler_params=pltpu.CompilerParams(needs_layout_passes=False),
    )(x_flat)


def kernel(x):
    shape = x.shape
    t = shape[-1]
    assert t % _PERIOD == 0
    n_out_t = t // _PERIOD
    n_out = x.size // _PERIOD
    y = _decimate_flat(x.reshape(-1), n_out)
    return y.reshape(*shape[:-1], n_out_t)
